# baseline (speedup 1.0000x reference)
import functools

import jax
import jax.numpy as jnp
from jax import lax
from jax.experimental import pallas as pl
from jax.experimental.pallas import tpu as pltpu
from jax.experimental.pallas import tpu_sc as plsc

BATCH = 16384
HIST = 50
LENGTH = 64
B = BATCH * HIST
NW = 32
BPW = B // NW
G = 128
NG = BPW // G
CW = 144

_mesh = plsc.VectorSubcoreMesh(core_axis_name="c", subcore_axis_name="s")


@functools.partial(
    pl.kernel,
    mesh=_mesh,
    out_type=[
        jax.ShapeDtypeStruct((B, 2 * LENGTH), jnp.float32),
        jax.ShapeDtypeStruct((B, 2), jnp.float32),
    ],
    scratch_types=[
        pltpu.VMEM((NG, G), jnp.int32),
        pltpu.VMEM((G, CW), jnp.float32),
        pltpu.VMEM((G, CW), jnp.float32),
        pltpu.SemaphoreType.DMA,
        pltpu.SemaphoreType.DMA,
        pltpu.SemaphoreType.DMA,
    ],
    compiler_params=pltpu.CompilerParams(use_tc_tiling_on_sc=False),
)
def _gather3(idx_hbm, combo_hbm,
             comb_out, dir_out,
             idx_v, buf_a, buf_b,
             gsem_a, gsem_b, wsem):
    wid = lax.axis_index("s") * 2 + lax.axis_index("c")
    base = wid * BPW
    pltpu.sync_copy(idx_hbm.at[wid], idx_v)

    bufs = (buf_a, buf_b)
    gsems = (gsem_a, gsem_b)

    def fire_gather(b, g):
        pltpu.async_copy(combo_hbm.at[idx_v.at[g]], bufs[b], gsems[b])

    def drain_gather(b, g):
        pltpu.make_async_copy(combo_hbm.at[idx_v.at[g]], bufs[b], gsems[b]).wait()

    def write_list(b, g):
        off = g * G
        return (
            (bufs[b].at[:, pl.ds(0, 2 * LENGTH)], comb_out.at[pl.ds(base + off, G)]),
            (bufs[b].at[:, pl.ds(2 * LENGTH, 2)], dir_out.at[pl.ds(base + off, G)]),
        )

    def fire_writes(b, g):
        for src, dst in write_list(b, g):
            pltpu.async_copy(src, dst, wsem)

    def drain_writes(b, g):
        for src, dst in write_list(b, g):
            pltpu.make_async_copy(src, dst, wsem).wait()

    fire_gather(0, 0)
    fire_gather(1, 1)

    @pl.loop(0, NG - 2, step=2)
    def _steady(g0):
        for b in range(2):
            g = g0 + b
            drain_gather(b, g)
            fire_writes(b, g)
            drain_writes(b, g)
            fire_gather(b, g + 2)

    for b, g in ((0, NG - 2), (1, NG - 1)):
        drain_gather(b, g)
        fire_writes(b, g)
        drain_writes(b, g)


def kernel(input, row_table, col_table, dir_table):
    idx = input.reshape(NW, NG, G).astype(jnp.int32)
    combo = jnp.concatenate(
        [row_table, col_table, dir_table,
         jnp.zeros((row_table.shape[0], CW - 2 * LENGTH - 2), jnp.float32)],
        axis=1)
    comb, dir_ = _gather3(idx, combo)
    return (
        comb[:, :LENGTH].reshape(BATCH, HIST, LENGTH),
        comb[:, LENGTH:].reshape(BATCH, HIST, LENGTH),
        dir_.reshape(BATCH, HIST, 2),
    )


# speedup vs baseline: 1.2049x; 1.2049x over previous
import functools

import jax
import jax.numpy as jnp
from jax import lax
from jax.experimental import pallas as pl
from jax.experimental.pallas import tpu as pltpu
from jax.experimental.pallas import tpu_sc as plsc

BATCH = 16384
HIST = 50
LENGTH = 64
B = BATCH * HIST
NW = 32
BPW = B // NW
G = 128
NG = BPW // G
DIRW = 16

_mesh = plsc.VectorSubcoreMesh(core_axis_name="c", subcore_axis_name="s")


@functools.partial(
    pl.kernel,
    mesh=_mesh,
    out_type=[
        jax.ShapeDtypeStruct((B, LENGTH), jnp.float32),
        jax.ShapeDtypeStruct((B, LENGTH), jnp.float32),
        jax.ShapeDtypeStruct((B, 2), jnp.float32),
    ],
    scratch_types=[
        pltpu.VMEM((NG, G), jnp.int32),
        pltpu.VMEM((G, LENGTH), jnp.float32),
        pltpu.VMEM((G, LENGTH), jnp.float32),
        pltpu.VMEM((G, DIRW), jnp.float32),
        pltpu.VMEM((G, LENGTH), jnp.float32),
        pltpu.VMEM((G, LENGTH), jnp.float32),
        pltpu.VMEM((G, DIRW), jnp.float32),
        pltpu.SemaphoreType.DMA,
        pltpu.SemaphoreType.DMA,
        pltpu.SemaphoreType.DMA,
    ],
    compiler_params=pltpu.CompilerParams(use_tc_tiling_on_sc=False),
)
def _gather3(idx_hbm, row_hbm, col_hbm, dir_hbm,
             row_out, col_out, dir_out,
             idx_v, row_a, col_a, dir_a, row_b, col_b, dir_b,
             gsem_a, gsem_b, wsem):
    wid = lax.axis_index("s") * 2 + lax.axis_index("c")
    base = wid * BPW
    pltpu.sync_copy(idx_hbm.at[wid], idx_v)

    row_bufs = (row_a, row_b)
    col_bufs = (col_a, col_b)
    dir_bufs = (dir_a, dir_b)
    gsems = (gsem_a, gsem_b)

    def fire_gathers(b, g):
        idxs = idx_v.at[g]
        pltpu.async_copy(row_hbm.at[idxs], row_bufs[b], gsems[b])
        pltpu.async_copy(col_hbm.at[idxs], col_bufs[b], gsems[b])
        pltpu.async_copy(dir_hbm.at[idxs], dir_bufs[b], gsems[b])

    def drain_gathers(b, g):
        idxs = idx_v.at[g]
        pltpu.make_async_copy(row_hbm.at[idxs], row_bufs[b], gsems[b]).wait()
        pltpu.make_async_copy(col_hbm.at[idxs], col_bufs[b], gsems[b]).wait()
        pltpu.make_async_copy(dir_hbm.at[idxs], dir_bufs[b], gsems[b]).wait()

    def fire_writes(b, g):
        off = g * G
        pltpu.async_copy(row_bufs[b], row_out.at[pl.ds(base + off, G)], wsem)
        pltpu.async_copy(col_bufs[b], col_out.at[pl.ds(base + off, G)], wsem)
        pltpu.async_copy(dir_bufs[b].at[:, pl.ds(0, 2)],
                         dir_out.at[pl.ds(base + off, G)], wsem)

    def drain_writes(b, g):
        off = g * G
        pltpu.make_async_copy(row_bufs[b], row_out.at[pl.ds(base + off, G)], wsem).wait()
        pltpu.make_async_copy(col_bufs[b], col_out.at[pl.ds(base + off, G)], wsem).wait()
        pltpu.make_async_copy(dir_bufs[b].at[:, pl.ds(0, 2)],
                              dir_out.at[pl.ds(base + off, G)], wsem).wait()

    fire_gathers(0, 0)
    fire_gathers(1, 1)

    @pl.loop(0, NG - 2, step=2)
    def _steady(g0):
        for b in range(2):
            g = g0 + b
            drain_gathers(b, g)
            fire_writes(b, g)
            drain_writes(b, g)
            fire_gathers(b, g + 2)

    for b, g in ((0, NG - 2), (1, NG - 1)):
        drain_gathers(b, g)
        fire_writes(b, g)
        drain_writes(b, g)


def kernel(input, row_table, col_table, dir_table):
    idx = input.reshape(NW, NG, G).astype(jnp.int32)
    dir_wide = jnp.pad(dir_table, ((0, 0), (0, DIRW - 2)))
    row, col, dir_ = _gather3(idx, row_table, col_table, dir_wide)
    return (
        row.reshape(BATCH, HIST, LENGTH),
        col.reshape(BATCH, HIST, LENGTH),
        dir_.reshape(BATCH, HIST, 2),
    )
